# baseline (speedup 1.0000x reference)
import functools

import jax
import jax.numpy as jnp
from jax import lax
from jax.experimental import pallas as pl
from jax.experimental.pallas import tpu as pltpu
from jax.experimental.pallas import tpu_sc as plsc

VOCAB = 100000
D = 1024
B, L = 4, 2048
N = B * L
NW = 32
CHUNK = N // NW
S = 16
NSUB = CHUNK // S

_mesh = plsc.VectorSubcoreMesh(core_axis_name="c", subcore_axis_name="s")


@functools.partial(
    pl.kernel,
    out_type=[
        jax.ShapeDtypeStruct((N,), jnp.int32),
        jax.ShapeDtypeStruct((N,), jnp.int32),
        jax.ShapeDtypeStruct((N,), jnp.int32),
        jax.ShapeDtypeStruct((N,), jnp.int32),
        jax.ShapeDtypeStruct((N,), jnp.int32),
        jax.ShapeDtypeStruct((N, D), jnp.float32),
    ],
    mesh=_mesh,
    scratch_types=[
        pltpu.VMEM((CHUNK,), jnp.int32),
        pltpu.VMEM((CHUNK,), jnp.int32),
        pltpu.VMEM((16,), jnp.int32),
        pltpu.VMEM((CHUNK,), jnp.int32),
        pltpu.VMEM((CHUNK,), jnp.int32),
        pltpu.VMEM((CHUNK,), jnp.int32),
        pltpu.VMEM((CHUNK,), jnp.int32),
        pltpu.VMEM((CHUNK,), jnp.int32),
        pltpu.VMEM((2, S, 16), jnp.float32),
        pltpu.VMEM((2, S, 16), jnp.float32),
        pltpu.VMEM((S, D), jnp.float32),
        pltpu.VMEM((S, D), jnp.float32),
        pltpu.VMEM((S, D), jnp.float32),
        pltpu.VMEM((S, D), jnp.float32),
        pltpu.VMEM((S,), jnp.int32),
        pltpu.VMEM((S,), jnp.int32),
        pltpu.VMEM((S,), jnp.int32),
        pltpu.VMEM((S,), jnp.int32),
        pltpu.SemaphoreType.DMA,
        pltpu.SemaphoreType.DMA,
        pltpu.SemaphoreType.DMA,
        pltpu.SemaphoreType.DMA,
        pltpu.SemaphoreType.DMA,
        pltpu.SemaphoreType.DMA,
        pltpu.SemaphoreType.DMA,
        pltpu.SemaphoreType.DMA,
        pltpu.SemaphoreType.DMA,
        pltpu.SemaphoreType.DMA,
        pltpu.SemaphoreType.DMA,
        pltpu.SemaphoreType.DMA,
    ],
)
def _emb_kernel(ids_h, vids_h, len_h, sca_h, scb_h, llm_h, code_h,
                mll_h, mcd_h, lin_h, cin_h, att_h, emb_h,
                ids_v, vids_v, len_v, mll_v, mcd_v, lin_v, cin_v, att_v,
                mba, mbb,
                rows_a0, rows_b0, rows_a1, rows_b1,
                idx_a0, idx_b0, idx_a1, idx_b1,
                sem_ga0, sem_gb0, sem_ga1, sem_gb1, sem_w0, sem_w1,
                sem_s0, sem_s1,
                sem_ga0b, sem_gb0b, sem_ga1b, sem_gb1b):
    c = lax.axis_index("c")
    s = lax.axis_index("s")
    wid = s * 2 + c
    base = wid * CHUNK

    pltpu.sync_copy(ids_h.at[pl.ds(base, CHUNK)], ids_v)
    pltpu.sync_copy(vids_h.at[pl.ds(base, CHUNK)], vids_v)
    pltpu.sync_copy(len_h.at[wid], len_v)

    iot = lax.iota(jnp.int32, 16)
    lenb = len_v[...]
    pos_base = base % L

    rows_a = (rows_a0, rows_a1)
    rows_b = (rows_b0, rows_b1)
    idx_a = (idx_a0, idx_a1)
    idx_b = (idx_b0, idx_b1)
    sem_ga = (sem_ga0, sem_ga1)
    sem_gb = (sem_gb0, sem_gb1)
    sem_w = (sem_w0, sem_w1)
    sem_s = (sem_s0, sem_s1)
    sem_ga2 = (sem_ga0b, sem_ga1b)
    sem_gb2 = (sem_gb0b, sem_gb1b)

    def fire(j):
        p = j & 1
        sl16 = pl.ds(j * S, 16)
        ids = ids_v[sl16]
        vid = vids_v[sl16]
        pos = pos_base + j * S + iot
        one = jnp.ones((16,), jnp.int32)
        zero = jnp.zeros((16,), jnp.int32)
        attn = jnp.where(pos < lenb, one, zero)
        m_cod = attn * vid
        m_llm = attn - m_cod
        llm_idx = ids * m_llm
        cod_idx = ids * m_cod
        mll_v[sl16] = m_llm
        mcd_v[sl16] = m_cod
        att_v[sl16] = attn
        lin_v[sl16] = llm_idx
        cin_v[sl16] = cod_idx
        tglob = base + j * S + iot
        idx_a[p][...] = llm_idx + tglob * (1 - m_llm)
        idx_b[p][...] = cod_idx + tglob * (1 - m_cod)
        h = S // 2
        cp_a = pltpu.async_copy(
            llm_h.at[idx_a[p].at[pl.ds(0, h)]],
            rows_a[p].at[pl.ds(0, h)], sem_ga[p])
        cp_a2 = pltpu.async_copy(
            llm_h.at[idx_a[p].at[pl.ds(h, h)]],
            rows_a[p].at[pl.ds(h, h)], sem_ga2[p])
        cp_b = pltpu.async_copy(
            code_h.at[idx_b[p].at[pl.ds(0, h)]],
            rows_b[p].at[pl.ds(0, h)], sem_gb[p])
        cp_b2 = pltpu.async_copy(
            code_h.at[idx_b[p].at[pl.ds(h, h)]],
            rows_b[p].at[pl.ds(h, h)], sem_gb2[p])
        cp_c = pltpu.async_copy(
            sca_h.at[pl.ds(base + j * S, S)], mba.at[p], sem_s[p])
        cp_d = pltpu.async_copy(
            scb_h.at[pl.ds(base + j * S, S)], mbb.at[p], sem_s[p])
        return cp_a, cp_a2, cp_b, cp_b2, cp_c, cp_d

    pending_g = {0: fire(0)}
    pending_w = {}
    for j in range(NSUB):
        p = j & 1
        if j + 1 < NSUB:
            if j - 1 in pending_w:
                pending_w.pop(j - 1).wait()
            pending_g[j + 1] = fire(j + 1)
        for cp in pending_g.pop(j):
            cp.wait()

        ra, rb = rows_a[p], rows_b[p]

        def addbody(i, carry2, p=p, ra=ra, rb=rb):
            r = i >> 3
            cb = (i & 7) * 128
            mla = mba[p, r, :]
            mcb = mbb[p, r, :]
            for k in range(8):
                csl = pl.ds(cb + k * 16, 16)
                ra[r, csl] = ra[r, csl] * mla + rb[r, csl] * mcb
            return carry2

        lax.fori_loop(0, S * 8, addbody, 0)
        pending_w[j] = pltpu.async_copy(
            ra, emb_h.at[pl.ds(base + j * S, S)], sem_w[p])
    for j in sorted(pending_w):
        pending_w.pop(j).wait()

    pltpu.sync_copy(mll_v, mll_h.at[pl.ds(base, CHUNK)])
    pltpu.sync_copy(mcd_v, mcd_h.at[pl.ds(base, CHUNK)])
    pltpu.sync_copy(lin_v, lin_h.at[pl.ds(base, CHUNK)])
    pltpu.sync_copy(cin_v, cin_h.at[pl.ds(base, CHUNK)])
    pltpu.sync_copy(att_v, att_h.at[pl.ds(base, CHUNK)])


def kernel(input_ids, vocab_ids, length, llm_table, code_table):
    ids = input_ids.reshape(-1)
    vids = vocab_ids.reshape(-1)
    len_bcast = jnp.broadcast_to(
        jnp.repeat(length, NW // B)[:, None], (NW, 16))
    attn_f = (jnp.arange(L)[None, :] < length[:, None])
    sca = ((vocab_ids == 0) & attn_f).astype(jnp.float32).reshape(-1)
    scb = ((vocab_ids == 1) & attn_f).astype(jnp.float32).reshape(-1)
    sca = jnp.broadcast_to(sca[:, None], (N, 16))
    scb = jnp.broadcast_to(scb[:, None], (N, 16))
    mll, mcd, lin, cin, att, emb = _emb_kernel(
        ids, vids, len_bcast, sca, scb, llm_table, code_table)
    shp = (B, L)
    return (mll.reshape(shp).astype(bool),
            mcd.reshape(shp).astype(bool),
            lin.reshape(shp),
            cin.reshape(shp),
            att.reshape(shp).astype(bool),
            emb.reshape(B, L, D))


# speedup vs baseline: 1.0815x; 1.0815x over previous
import functools

import jax
import jax.numpy as jnp
from jax import lax
from jax.experimental import pallas as pl
from jax.experimental.pallas import tpu as pltpu
from jax.experimental.pallas import tpu_sc as plsc

VOCAB = 100000
D = 1024
B, L = 4, 2048
N = B * L
NW = 32
CHUNK = N // NW
S = 16
NSUB = CHUNK // S

_mesh = plsc.VectorSubcoreMesh(core_axis_name="c", subcore_axis_name="s")


@functools.partial(
    pl.kernel,
    out_type=[
        jax.ShapeDtypeStruct((N,), jnp.int32),
        jax.ShapeDtypeStruct((N,), jnp.int32),
        jax.ShapeDtypeStruct((N,), jnp.int32),
        jax.ShapeDtypeStruct((N,), jnp.int32),
        jax.ShapeDtypeStruct((N,), jnp.int32),
        jax.ShapeDtypeStruct((B, L, D), jnp.float32),
    ],
    mesh=_mesh,
    scratch_types=[
        pltpu.VMEM((CHUNK,), jnp.int32),
        pltpu.VMEM((CHUNK,), jnp.int32),
        pltpu.VMEM((16,), jnp.int32),
        pltpu.VMEM((CHUNK,), jnp.int32),
        pltpu.VMEM((CHUNK,), jnp.int32),
        pltpu.VMEM((CHUNK,), jnp.int32),
        pltpu.VMEM((CHUNK,), jnp.int32),
        pltpu.VMEM((CHUNK,), jnp.int32),
        pltpu.VMEM((2, S, 16), jnp.float32),
        pltpu.VMEM((S, D), jnp.float32),
        pltpu.VMEM((S, D), jnp.float32),
        pltpu.VMEM((S, D), jnp.float32),
        pltpu.VMEM((S, D), jnp.float32),
        pltpu.VMEM((S,), jnp.int32),
        pltpu.VMEM((S,), jnp.int32),
        pltpu.VMEM((S,), jnp.int32),
        pltpu.VMEM((S,), jnp.int32),
        pltpu.SemaphoreType.DMA,
        pltpu.SemaphoreType.DMA,
        pltpu.SemaphoreType.DMA,
        pltpu.SemaphoreType.DMA,
        pltpu.SemaphoreType.DMA,
        pltpu.SemaphoreType.DMA,
        pltpu.SemaphoreType.DMA,
        pltpu.SemaphoreType.DMA,
    ],
)
def _emb_kernel(ids_h, vids_h, len_h, wgt_h, llm_h, code_h,
                mll_h, mcd_h, lin_h, cin_h, att_h, emb_h,
                ids_v, vids_v, len_v, mll_v, mcd_v, lin_v, cin_v, att_v,
                mbw,
                rows_a0, rows_b0, rows_a1, rows_b1,
                idx_a0, idx_b0, idx_a1, idx_b1,
                sem_ga0, sem_gb0, sem_ga1, sem_gb1, sem_w0, sem_w1,
                sem_s0, sem_s1):
    c = lax.axis_index("c")
    s = lax.axis_index("s")
    wid = s * 2 + c
    base = wid * CHUNK

    pltpu.sync_copy(ids_h.at[pl.ds(base, CHUNK)], ids_v)
    pltpu.sync_copy(vids_h.at[pl.ds(base, CHUNK)], vids_v)
    pltpu.sync_copy(len_h.at[wid], len_v)

    iot = lax.iota(jnp.int32, 16)
    lenb = len_v[...]
    pos_base = base % L

    rows_a = (rows_a0, rows_a1)
    rows_b = (rows_b0, rows_b1)
    idx_a = (idx_a0, idx_a1)
    idx_b = (idx_b0, idx_b1)
    sem_ga = (sem_ga0, sem_ga1)
    sem_gb = (sem_gb0, sem_gb1)
    sem_w = (sem_w0, sem_w1)
    sem_s = (sem_s0, sem_s1)

    def fire(j):
        p = j & 1
        sl16 = pl.ds(j * S, 16)
        ids = ids_v[sl16]
        vid = vids_v[sl16]
        pos = pos_base + j * S + iot
        one = jnp.ones((16,), jnp.int32)
        zero = jnp.zeros((16,), jnp.int32)
        attn = jnp.where(pos < lenb, one, zero)
        m_cod = attn * vid
        m_llm = attn - m_cod
        llm_idx = ids * m_llm
        cod_idx = ids * m_cod
        mll_v[sl16] = m_llm
        mcd_v[sl16] = m_cod
        att_v[sl16] = attn
        lin_v[sl16] = llm_idx
        cin_v[sl16] = cod_idx
        tglob = base + j * S + iot
        idx_a[p][...] = llm_idx + tglob * (1 - m_llm)
        idx_b[p][...] = cod_idx + tglob * (1 - m_cod)
        cp_a = pltpu.async_copy(llm_h.at[idx_a[p]], rows_a[p], sem_ga[p])
        cp_b = pltpu.async_copy(code_h.at[idx_b[p]], rows_b[p], sem_gb[p])
        cp_c = pltpu.async_copy(
            wgt_h.at[pl.ds(base + j * S, S)], mbw.at[p], sem_s[p])
        return cp_a, cp_b, cp_c

    pending_g = {0: fire(0)}
    pending_w = {}
    for j in range(NSUB):
        p = j & 1
        if j + 1 < NSUB:
            if j - 1 in pending_w:
                pending_w.pop(j - 1).wait()
            pending_g[j + 1] = fire(j + 1)
        for cp in pending_g.pop(j):
            cp.wait()

        ra, rb = rows_a[p], rows_b[p]

        def addbody(i, carry2, p=p, ra=ra, rb=rb):
            r = i >> 3
            cb = (i & 7) * 128
            w = mbw[p, r, :]
            mla = w * (2.0 - w)
            mcb = w * (w - 1.0) * 0.5
            for k in range(8):
                csl = pl.ds(cb + k * 16, 16)
                ra[r, csl] = ra[r, csl] * mla + rb[r, csl] * mcb
            return carry2

        lax.fori_loop(0, S * 8, addbody, 0)
        pending_w[j] = pltpu.async_copy(
            ra, emb_h.at[base // L, pl.ds(pos_base + j * S, S)], sem_w[p])
    for j in sorted(pending_w):
        pending_w.pop(j).wait()

    pltpu.sync_copy(mll_v, mll_h.at[pl.ds(base, CHUNK)])
    pltpu.sync_copy(mcd_v, mcd_h.at[pl.ds(base, CHUNK)])
    pltpu.sync_copy(lin_v, lin_h.at[pl.ds(base, CHUNK)])
    pltpu.sync_copy(cin_v, cin_h.at[pl.ds(base, CHUNK)])
    pltpu.sync_copy(att_v, att_h.at[pl.ds(base, CHUNK)])


def kernel(input_ids, vocab_ids, length, llm_table, code_table):
    ids = input_ids.reshape(-1)
    vids = vocab_ids.reshape(-1)
    len_bcast = jnp.broadcast_to(
        jnp.repeat(length, NW // B)[:, None], (NW, 16))
    attn_f = (jnp.arange(L)[None, :] < length[:, None])
    wgt = jnp.where(attn_f, vocab_ids + 1, 0).astype(jnp.float32).reshape(-1)
    wgt = jnp.broadcast_to(wgt[:, None], (N, 16))
    mll, mcd, lin, cin, att, emb = _emb_kernel(
        ids, vids, len_bcast, wgt, llm_table, code_table)
    shp = (B, L)
    return (mll.reshape(shp).astype(bool),
            mcd.reshape(shp).astype(bool),
            lin.reshape(shp),
            cin.reshape(shp),
            att.reshape(shp).astype(bool),
            emb)


# speedup vs baseline: 1.0969x; 1.0142x over previous
import functools

import jax
import jax.numpy as jnp
from jax import lax
from jax.experimental import pallas as pl
from jax.experimental.pallas import tpu as pltpu
from jax.experimental.pallas import tpu_sc as plsc

VOCAB = 100000
D = 1024
B, L = 4, 2048
N = B * L
NW = 32
CHUNK = N // NW
S = 16
NSUB = CHUNK // S

_mesh = plsc.VectorSubcoreMesh(core_axis_name="c", subcore_axis_name="s")


@functools.partial(
    pl.kernel,
    out_type=[
        jax.ShapeDtypeStruct((N,), jnp.int32),
        jax.ShapeDtypeStruct((N,), jnp.int32),
        jax.ShapeDtypeStruct((N,), jnp.int32),
        jax.ShapeDtypeStruct((N,), jnp.int32),
        jax.ShapeDtypeStruct((N,), jnp.int32),
        jax.ShapeDtypeStruct((B, L, D), jnp.float32),
    ],
    mesh=_mesh,
    scratch_types=[
        pltpu.VMEM((CHUNK,), jnp.int32),
        pltpu.VMEM((CHUNK,), jnp.int32),
        pltpu.VMEM((16,), jnp.int32),
        pltpu.VMEM((CHUNK,), jnp.int32),
        pltpu.VMEM((CHUNK,), jnp.int32),
        pltpu.VMEM((CHUNK,), jnp.int32),
        pltpu.VMEM((CHUNK,), jnp.int32),
        pltpu.VMEM((CHUNK,), jnp.int32),
        pltpu.VMEM((2, S, 16), jnp.float32),
        pltpu.VMEM((S, D), jnp.float32),
        pltpu.VMEM((S, D), jnp.float32),
        pltpu.VMEM((S, D), jnp.float32),
        pltpu.VMEM((S, D), jnp.float32),
        pltpu.VMEM((S,), jnp.int32),
        pltpu.VMEM((S,), jnp.int32),
        pltpu.VMEM((S,), jnp.int32),
        pltpu.VMEM((S,), jnp.int32),
        pltpu.SemaphoreType.DMA,
        pltpu.SemaphoreType.DMA,
        pltpu.SemaphoreType.DMA,
        pltpu.SemaphoreType.DMA,
        pltpu.SemaphoreType.DMA,
        pltpu.SemaphoreType.DMA,
        pltpu.SemaphoreType.DMA,
        pltpu.SemaphoreType.DMA,
    ],
)
def _emb_kernel(ids_h, vids_h, len_h, wgt_h, llm_h, code_h,
                mll_h, mcd_h, lin_h, cin_h, att_h, emb_h,
                ids_v, vids_v, len_v, mll_v, mcd_v, lin_v, cin_v, att_v,
                mbw,
                rows_a0, rows_b0, rows_a1, rows_b1,
                idx_a0, idx_b0, idx_a1, idx_b1,
                sem_ga0, sem_gb0, sem_ga1, sem_gb1, sem_w0, sem_w1,
                sem_s0, sem_s1):
    c = lax.axis_index("c")
    s = lax.axis_index("s")
    wid = s * 2 + c
    base = wid * CHUNK

    cp0 = pltpu.async_copy(ids_h.at[pl.ds(base, CHUNK)], ids_v, sem_ga0)
    cp1 = pltpu.async_copy(vids_h.at[pl.ds(base, CHUNK)], vids_v, sem_gb0)
    cp2 = pltpu.async_copy(len_h.at[wid], len_v, sem_s0)
    cp0.wait()
    cp1.wait()
    cp2.wait()

    iot = lax.iota(jnp.int32, 16)
    lenb = len_v[...]
    pos_base = base % L

    rows_a = (rows_a0, rows_a1)
    rows_b = (rows_b0, rows_b1)
    idx_a = (idx_a0, idx_a1)
    idx_b = (idx_b0, idx_b1)
    sem_ga = (sem_ga0, sem_ga1)
    sem_gb = (sem_gb0, sem_gb1)
    sem_w = (sem_w0, sem_w1)
    sem_s = (sem_s0, sem_s1)

    def fire(j):
        p = j & 1
        sl16 = pl.ds(j * S, 16)
        ids = ids_v[sl16]
        vid = vids_v[sl16]
        pos = pos_base + j * S + iot
        one = jnp.ones((16,), jnp.int32)
        zero = jnp.zeros((16,), jnp.int32)
        attn = jnp.where(pos < lenb, one, zero)
        m_cod = attn * vid
        m_llm = attn - m_cod
        llm_idx = ids * m_llm
        cod_idx = ids * m_cod
        mll_v[sl16] = m_llm
        mcd_v[sl16] = m_cod
        att_v[sl16] = attn
        lin_v[sl16] = llm_idx
        cin_v[sl16] = cod_idx
        tglob = base + j * S + iot
        idx_a[p][...] = llm_idx + tglob * (1 - m_llm)
        idx_b[p][...] = cod_idx + tglob * (1 - m_cod)
        cp_a = pltpu.async_copy(llm_h.at[idx_a[p]], rows_a[p], sem_ga[p])
        cp_b = pltpu.async_copy(code_h.at[idx_b[p]], rows_b[p], sem_gb[p])
        cp_c = pltpu.async_copy(
            wgt_h.at[pl.ds(base + j * S, S)], mbw.at[p], sem_s[p])
        return cp_a, cp_b, cp_c

    pending_g = {0: fire(0)}
    pending_w = {}
    for j in range(NSUB):
        p = j & 1
        if j + 1 < NSUB:
            if j - 1 in pending_w:
                pending_w.pop(j - 1).wait()
            pending_g[j + 1] = fire(j + 1)
        for cp in pending_g.pop(j):
            cp.wait()

        ra, rb = rows_a[p], rows_b[p]

        def addbody(i, carry2, p=p, ra=ra, rb=rb):
            r = i >> 3
            cb = (i & 7) * 128
            w = mbw[p, r, :]
            mla = w * (2.0 - w)
            mcb = w * (w - 1.0) * 0.5
            for k in range(8):
                csl = pl.ds(cb + k * 16, 16)
                ra[r, csl] = ra[r, csl] * mla + rb[r, csl] * mcb
            return carry2

        lax.fori_loop(0, S * 8, addbody, 0)
        pending_w[j] = pltpu.async_copy(
            ra, emb_h.at[base // L, pl.ds(pos_base + j * S, S)], sem_w[p])
    cs0 = pltpu.async_copy(mll_v, mll_h.at[pl.ds(base, CHUNK)], sem_ga0)
    cs1 = pltpu.async_copy(mcd_v, mcd_h.at[pl.ds(base, CHUNK)], sem_gb0)
    cs2 = pltpu.async_copy(lin_v, lin_h.at[pl.ds(base, CHUNK)], sem_ga1)
    cs3 = pltpu.async_copy(cin_v, cin_h.at[pl.ds(base, CHUNK)], sem_gb1)
    cs4 = pltpu.async_copy(att_v, att_h.at[pl.ds(base, CHUNK)], sem_s0)
    for j in sorted(pending_w):
        pending_w.pop(j).wait()
    for cs in (cs0, cs1, cs2, cs3, cs4):
        cs.wait()


def kernel(input_ids, vocab_ids, length, llm_table, code_table):
    ids = input_ids.reshape(-1)
    vids = vocab_ids.reshape(-1)
    len_bcast = jnp.broadcast_to(
        jnp.repeat(length, NW // B)[:, None], (NW, 16))
    attn_f = (jnp.arange(L)[None, :] < length[:, None])
    wgt = jnp.where(attn_f, vocab_ids + 1, 0).astype(jnp.float32).reshape(-1)
    wgt = jnp.broadcast_to(wgt[:, None], (N, 16))
    mll, mcd, lin, cin, att, emb = _emb_kernel(
        ids, vids, len_bcast, wgt, llm_table, code_table)
    shp = (B, L)
    return (mll.reshape(shp).astype(bool),
            mcd.reshape(shp).astype(bool),
            lin.reshape(shp),
            cin.reshape(shp),
            att.reshape(shp).astype(bool),
            emb)
